# Initial kernel scaffold; baseline (speedup 1.0000x reference)
import jax
import jax.numpy as jnp
from jax.experimental import pallas as pl


def kernel(pos, edge_index, batch, W1, a_src1, a_dst1, b1, in1_w, in1_b, pool_w, W2, a_src2, a_dst2, b2, in2_w, in2_b, gw1, gb1, gw2, gb2, nw, nb, l1w, l1b, l2w, l2b, l3w, l3b):
    raise NotImplementedError("write your pallas kernel here")


# speedup vs baseline: 1.1828x; 1.1828x over previous
import functools
import jax
import jax.numpy as jnp
from jax import lax
from jax.experimental import pallas as pl

_N = 50000
_G = 64
_NC = 10
_RATIO = 0.3
_BN = 1000
_INTERPRET = False


def _elu(x):
    return jnp.where(x > 0, x, jnp.exp(jnp.minimum(x, 0.0)) - 1.0)


def _proj_body(x_ref, w_ref, asrc_ref, adst_ref, h_ref, als_ref, ald_ref):
    x = x_ref[...]
    h = jnp.dot(x, w_ref[...], preferred_element_type=jnp.float32)
    h_ref[...] = h
    als_ref[...] = jnp.dot(h, asrc_ref[...], preferred_element_type=jnp.float32)
    ald_ref[...] = jnp.dot(h, adst_ref[...], preferred_element_type=jnp.float32)


def _node_proj(x, W, Asrc, Adst):
    n, cin = x.shape
    cout = W.shape[1]
    grid = n // _BN
    return pl.pallas_call(
        _proj_body,
        grid=(grid,),
        in_specs=[
            pl.BlockSpec((_BN, cin), lambda i: (i, 0)),
            pl.BlockSpec((cin, cout), lambda i: (0, 0)),
            pl.BlockSpec((cout, 2), lambda i: (0, 0)),
            pl.BlockSpec((cout, 2), lambda i: (0, 0)),
        ],
        out_specs=[
            pl.BlockSpec((_BN, cout), lambda i: (i, 0)),
            pl.BlockSpec((_BN, 2), lambda i: (i, 0)),
            pl.BlockSpec((_BN, 2), lambda i: (i, 0)),
        ],
        out_shape=[
            jax.ShapeDtypeStruct((n, cout), jnp.float32),
            jax.ShapeDtypeStruct((n, 2), jnp.float32),
            jax.ShapeDtypeStruct((n, 2), jnp.float32),
        ],
        interpret=_INTERPRET,
    )(x, W, Asrc, Adst)


def _stats_body(agg_ref, b_ref, batch_ref, y_ref, s1_ref, s2_ref, cnt_ref):
    i = pl.program_id(0)
    y = _elu(agg_ref[...] + b_ref[...])
    y_ref[...] = y
    bb = batch_ref[...]
    oh = (bb == lax.broadcasted_iota(jnp.int32, (1, _G), 1)).astype(jnp.float32)
    s1 = lax.dot_general(oh, y, (((0,), (0,)), ((), ())),
                         preferred_element_type=jnp.float32)
    s2 = lax.dot_general(oh, y * y, (((0,), (0,)), ((), ())),
                         preferred_element_type=jnp.float32)
    cnt = jnp.sum(oh, axis=0, keepdims=True)

    @pl.when(i == 0)
    def _init():
        s1_ref[...] = jnp.zeros_like(s1_ref)
        s2_ref[...] = jnp.zeros_like(s2_ref)
        cnt_ref[...] = jnp.zeros_like(cnt_ref)

    s1_ref[...] += s1
    s2_ref[...] += s2
    cnt_ref[...] += cnt


def _elu_stats(agg, b, batch2d):
    n, c = agg.shape
    grid = n // _BN
    return pl.pallas_call(
        _stats_body,
        grid=(grid,),
        in_specs=[
            pl.BlockSpec((_BN, c), lambda i: (i, 0)),
            pl.BlockSpec((1, c), lambda i: (0, 0)),
            pl.BlockSpec((_BN, 1), lambda i: (i, 0)),
        ],
        out_specs=[
            pl.BlockSpec((_BN, c), lambda i: (i, 0)),
            pl.BlockSpec((_G, c), lambda i: (0, 0)),
            pl.BlockSpec((_G, c), lambda i: (0, 0)),
            pl.BlockSpec((1, _G), lambda i: (0, 0)),
        ],
        out_shape=[
            jax.ShapeDtypeStruct((n, c), jnp.float32),
            jax.ShapeDtypeStruct((_G, c), jnp.float32),
            jax.ShapeDtypeStruct((_G, c), jnp.float32),
            jax.ShapeDtypeStruct((1, _G), jnp.float32),
        ],
        interpret=_INTERPRET,
    )(agg, b, batch2d)


def _norm_body(y_ref, mean_ref, rstd_ref, w_ref, b_ref, pw_ref, batch_ref,
               xn_ref, sc_ref):
    bb = batch_ref[...]
    oh = (bb == lax.broadcasted_iota(jnp.int32, (1, _G), 1)).astype(jnp.float32)
    mean = lax.dot_general(oh, mean_ref[...], (((1,), (0,)), ((), ())),
                           preferred_element_type=jnp.float32)
    rstd = lax.dot_general(oh, rstd_ref[...], (((1,), (0,)), ((), ())),
                           preferred_element_type=jnp.float32)
    xn = (y_ref[...] - mean) * rstd * w_ref[...] + b_ref[...]
    xn_ref[...] = xn
    pw = pw_ref[...]
    sc_ref[...] = jnp.tanh(jnp.dot(xn, pw, preferred_element_type=jnp.float32))


def _inorm_apply(y, meanG, rstdG, w, b, pw_scaled, batch2d):
    n, c = y.shape
    grid = n // _BN
    return pl.pallas_call(
        _norm_body,
        grid=(grid,),
        in_specs=[
            pl.BlockSpec((_BN, c), lambda i: (i, 0)),
            pl.BlockSpec((_G, c), lambda i: (0, 0)),
            pl.BlockSpec((_G, c), lambda i: (0, 0)),
            pl.BlockSpec((1, c), lambda i: (0, 0)),
            pl.BlockSpec((1, c), lambda i: (0, 0)),
            pl.BlockSpec((c, 1), lambda i: (0, 0)),
            pl.BlockSpec((_BN, 1), lambda i: (i, 0)),
        ],
        out_specs=[
            pl.BlockSpec((_BN, c), lambda i: (i, 0)),
            pl.BlockSpec((_BN, 1), lambda i: (i, 0)),
        ],
        out_shape=[
            jax.ShapeDtypeStruct((n, c), jnp.float32),
            jax.ShapeDtypeStruct((n, 1), jnp.float32),
        ],
        interpret=_INTERPRET,
    )(y, meanG, rstdG, w, b, pw_scaled, batch2d)


def _gate_body(xn_ref, gw1_ref, gb1_ref, gw2_ref, gb2_ref, nw_ref, nb_ref,
               batch_ref, gate_ref, xt_ref, m_ref):
    i = pl.program_id(0)
    xn = xn_ref[...]
    g1 = _elu(jnp.dot(xn, gw1_ref[...], preferred_element_type=jnp.float32)
              + gb1_ref[...])
    gate = jnp.dot(g1, gw2_ref[...], preferred_element_type=jnp.float32) \
        + gb2_ref[...]
    gate_ref[...] = gate
    xt_ref[...] = _elu(jnp.dot(xn, nw_ref[...],
                               preferred_element_type=jnp.float32) + nb_ref[...])
    bb = batch_ref[...]
    oh = bb == lax.broadcasted_iota(jnp.int32, (1, _G), 1)
    gm = jnp.max(jnp.where(oh, gate, -1e30), axis=0, keepdims=True)

    @pl.when(i == 0)
    def _init():
        m_ref[...] = jnp.full_like(m_ref, -1e30)

    m_ref[...] = jnp.maximum(m_ref[...], gm)


def _gate_head(xn, gw1, gb1, gw2, gb2, nw, nb, batch2d):
    n, c = xn.shape
    grid = n // _BN
    return pl.pallas_call(
        _gate_body,
        grid=(grid,),
        in_specs=[
            pl.BlockSpec((_BN, c), lambda i: (i, 0)),
            pl.BlockSpec((c, 128), lambda i: (0, 0)),
            pl.BlockSpec((1, 128), lambda i: (0, 0)),
            pl.BlockSpec((128, 1), lambda i: (0, 0)),
            pl.BlockSpec((1, 1), lambda i: (0, 0)),
            pl.BlockSpec((c, c), lambda i: (0, 0)),
            pl.BlockSpec((1, c), lambda i: (0, 0)),
            pl.BlockSpec((_BN, 1), lambda i: (i, 0)),
        ],
        out_specs=[
            pl.BlockSpec((_BN, 1), lambda i: (i, 0)),
            pl.BlockSpec((_BN, c), lambda i: (i, 0)),
            pl.BlockSpec((1, _G), lambda i: (0, 0)),
        ],
        out_shape=[
            jax.ShapeDtypeStruct((n, 1), jnp.float32),
            jax.ShapeDtypeStruct((n, c), jnp.float32),
            jax.ShapeDtypeStruct((1, _G), jnp.float32),
        ],
        interpret=_INTERPRET,
    )(xn, gw1, gb1, gw2, gb2, nw, nb, batch2d)


def _pool_body(gate_ref, xt_ref, m_ref, batch_ref, s_ref, t_ref):
    i = pl.program_id(0)
    bb = batch_ref[...]
    ohb = bb == lax.broadcasted_iota(jnp.int32, (1, _G), 1)
    oh = ohb.astype(jnp.float32)
    m = jnp.max(jnp.where(ohb, m_ref[...], -jnp.inf), axis=1, keepdims=True)
    m = jnp.where(jnp.isfinite(m), m, 0.0)
    e = jnp.exp(gate_ref[...] - m)
    s = jnp.sum(oh * e, axis=0, keepdims=True)
    t = lax.dot_general(oh * e, xt_ref[...], (((0,), (0,)), ((), ())),
                        preferred_element_type=jnp.float32)

    @pl.when(i == 0)
    def _init():
        s_ref[...] = jnp.zeros_like(s_ref)
        t_ref[...] = jnp.zeros_like(t_ref)

    s_ref[...] += s
    t_ref[...] += t


def _gate_pool(gate, xt, m, batch2d):
    n, c = xt.shape
    grid = n // _BN
    return pl.pallas_call(
        _pool_body,
        grid=(grid,),
        in_specs=[
            pl.BlockSpec((_BN, 1), lambda i: (i, 0)),
            pl.BlockSpec((_BN, c), lambda i: (i, 0)),
            pl.BlockSpec((1, _G), lambda i: (0, 0)),
            pl.BlockSpec((_BN, 1), lambda i: (i, 0)),
        ],
        out_specs=[
            pl.BlockSpec((1, _G), lambda i: (0, 0)),
            pl.BlockSpec((_G, c), lambda i: (0, 0)),
        ],
        out_shape=[
            jax.ShapeDtypeStruct((1, _G), jnp.float32),
            jax.ShapeDtypeStruct((_G, c), jnp.float32),
        ],
        interpret=_INTERPRET,
    )(gate, xt, m, batch2d)


def _head_body(t_ref, s_ref, w1_ref, b1_ref, w2_ref, b2_ref, w3_ref, b3_ref,
               out_ref):
    s = s_ref[...]
    xg = t_ref[...] / (s + 1e-16)
    h = _elu(jnp.dot(xg, w1_ref[...], preferred_element_type=jnp.float32)
             + b1_ref[...])
    h = _elu(jnp.dot(h, w2_ref[...], preferred_element_type=jnp.float32)
             + b2_ref[...])
    o = jnp.dot(h, w3_ref[...], preferred_element_type=jnp.float32) + b3_ref[...]
    mx = jnp.max(o, axis=1, keepdims=True)
    z = o - mx
    lse = jnp.log(jnp.sum(jnp.exp(z), axis=1, keepdims=True))
    out_ref[...] = z - lse


def _classifier(T, sg, l1w, l1b, l2w, l2b, l3w, l3b):
    return pl.pallas_call(
        _head_body,
        out_shape=jax.ShapeDtypeStruct((_G, _NC), jnp.float32),
        interpret=_INTERPRET,
    )(T, sg, l1w, l1b, l2w, l2b, l3w, l3b)


def _head_mat(a):
    heads, oc = a.shape
    cols = []
    for hh in range(heads):
        col = jnp.zeros((heads * oc,), jnp.float32).at[hh * oc:(hh + 1) * oc].set(a[hh])
        cols.append(col)
    return jnp.stack(cols, axis=1)


def _edge_aggregate(h, als, ald, s, d, n, heads, oc):
    al = als[s] + ald[d]
    al = jnp.where(al > 0, al, 0.2 * al)
    m = jax.ops.segment_max(al, d, num_segments=n)
    m = jnp.where(jnp.isfinite(m), m, 0.0)
    e = jnp.exp(al - m[d])
    ssum = jax.ops.segment_sum(e, d, num_segments=n)
    he = h[s].reshape(-1, heads, oc) * e[..., None]
    agg = jax.ops.segment_sum(he, d, num_segments=n)
    return (agg / (ssum + 1e-16)[..., None]).reshape(n, heads * oc)


def kernel(pos, edge_index, batch, W1, a_src1, a_dst1, b1, in1_w, in1_b,
           pool_w, W2, a_src2, a_dst2, b2, in2_w, in2_b, gw1, gb1, gw2, gb2,
           nw, nb, l1w, l1b, l2w, l2b, l3w, l3b):
    n = _N
    batch2d = batch.reshape(n, 1)

    keep = edge_index[0] != edge_index[1]
    s1 = jnp.where(keep, edge_index[0], n)
    d1 = jnp.where(keep, edge_index[1], n)
    loops = jnp.arange(n, dtype=jnp.int32)
    es = jnp.concatenate([s1, loops])
    ed = jnp.concatenate([d1, loops])

    h1, als1, ald1 = _node_proj(pos, W1, _head_mat(a_src1), _head_mat(a_dst1))
    agg1 = _edge_aggregate(h1, als1, ald1, es, ed, n, 2, 32)

    y1, S1, S2, cnt = _elu_stats(agg1, b1.reshape(1, -1), batch2d)
    cnt_g = jnp.maximum(cnt.reshape(_G, 1), 1.0)
    mean1 = S1 / cnt_g
    var1 = jnp.maximum(S2 / cnt_g - mean1 * mean1, 0.0)
    rstd1 = 1.0 / jnp.sqrt(var1 + 1e-5)
    pw = (pool_w / (jnp.linalg.norm(pool_w) + 1e-16)).reshape(-1, 1)
    x1, score = _inorm_apply(y1, mean1, rstd1, in1_w.reshape(1, -1),
                             in1_b.reshape(1, -1), pw, batch2d)
    score = score[:, 0]

    counts = cnt.reshape(_G)
    k = jnp.ceil(_RATIO * counts).astype(jnp.int32)
    order = jnp.lexsort((-score, batch))
    starts = jnp.concatenate([jnp.zeros((1,), jnp.int32),
                              jnp.cumsum(counts.astype(jnp.int32))[:-1]])
    pr = jnp.arange(n, dtype=jnp.int32) - starts[batch[order]]
    rank = jnp.zeros_like(pr).at[order].set(pr)
    mask = rank < k[batch]
    perm = jnp.nonzero(mask, size=n, fill_value=n)[0]
    m_nodes = jnp.sum(mask.astype(jnp.int32))
    valid = jnp.arange(n, dtype=jnp.int32) < m_nodes
    batch2 = jnp.where(valid, batch[perm], _G).astype(jnp.int32)
    emask = mask[edge_index[0]] & mask[edge_index[1]] & keep
    newid = jnp.cumsum(mask.astype(jnp.int32)) - 1
    src2 = jnp.where(emask, newid[edge_index[0]], n)
    dst2 = jnp.where(emask, newid[edge_index[1]], n)
    lid = jnp.where(valid, loops, n)
    es2 = jnp.concatenate([src2, lid])
    ed2 = jnp.concatenate([dst2, lid])

    xin2 = jnp.where(valid[:, None], x1[perm] * score[perm][:, None], 0.0)

    h2, als2, ald2 = _node_proj(xin2, W2, _head_mat(a_src2), _head_mat(a_dst2))
    agg2 = _edge_aggregate(h2, als2, ald2, es2, ed2, n, 2, 128)

    batch2d2 = batch2.reshape(n, 1)
    y2, S1b, S2b, cntb = _elu_stats(agg2, b2.reshape(1, -1), batch2d2)
    cnt_g2 = jnp.maximum(cntb.reshape(_G, 1), 1.0)
    mean2 = S1b / cnt_g2
    var2 = jnp.maximum(S2b / cnt_g2 - mean2 * mean2, 0.0)
    rstd2 = 1.0 / jnp.sqrt(var2 + 1e-5)
    zero_pw = jnp.zeros((256, 1), jnp.float32)
    x2, _ = _inorm_apply(y2, mean2, rstd2, in2_w.reshape(1, -1),
                         in2_b.reshape(1, -1), zero_pw, batch2d2)

    gate, xt, mg = _gate_head(x2, gw1, gb1.reshape(1, -1), gw2,
                              gb2.reshape(1, 1), nw, nb.reshape(1, -1),
                              batch2d2)
    sG, T = _gate_pool(gate, xt, mg, batch2d2)
    return _classifier(T, sG.reshape(_G, 1), l1w, l1b.reshape(1, -1),
                       l2w, l2b.reshape(1, -1), l3w, l3b.reshape(1, -1))


# speedup vs baseline: 4.3809x; 3.7040x over previous
import functools
import jax
import jax.numpy as jnp
from jax import lax
from jax.experimental import pallas as pl

_N = 50000
_G = 64
_NC = 10
_RATIO = 0.3
_BN = 1000
_INTERPRET = False


def _elu(x):
    return jnp.where(x > 0, x, jnp.exp(jnp.minimum(x, 0.0)) - 1.0)


def _proj_body(x_ref, w_ref, asrc_ref, adst_ref, h_ref, als_ref, ald_ref):
    x = x_ref[...]
    h = jnp.dot(x, w_ref[...], preferred_element_type=jnp.float32)
    h_ref[...] = h
    als_ref[...] = jnp.dot(h, asrc_ref[...], preferred_element_type=jnp.float32)
    ald_ref[...] = jnp.dot(h, adst_ref[...], preferred_element_type=jnp.float32)


def _node_proj(x, W, Asrc, Adst):
    n, cin = x.shape
    cout = W.shape[1]
    grid = n // _BN
    return pl.pallas_call(
        _proj_body,
        grid=(grid,),
        in_specs=[
            pl.BlockSpec((_BN, cin), lambda i: (i, 0)),
            pl.BlockSpec((cin, cout), lambda i: (0, 0)),
            pl.BlockSpec((cout, 2), lambda i: (0, 0)),
            pl.BlockSpec((cout, 2), lambda i: (0, 0)),
        ],
        out_specs=[
            pl.BlockSpec((_BN, cout), lambda i: (i, 0)),
            pl.BlockSpec((_BN, 2), lambda i: (i, 0)),
            pl.BlockSpec((_BN, 2), lambda i: (i, 0)),
        ],
        out_shape=[
            jax.ShapeDtypeStruct((n, cout), jnp.float32),
            jax.ShapeDtypeStruct((n, 2), jnp.float32),
            jax.ShapeDtypeStruct((n, 2), jnp.float32),
        ],
        interpret=_INTERPRET,
    )(x, W, Asrc, Adst)


def _stats_body(agg_ref, b_ref, batch_ref, y_ref, s1_ref, s2_ref, cnt_ref):
    i = pl.program_id(0)
    y = _elu(agg_ref[...] + b_ref[...])
    y_ref[...] = y
    bb = batch_ref[...]
    oh = (bb == lax.broadcasted_iota(jnp.int32, (1, _G), 1)).astype(jnp.float32)
    s1 = lax.dot_general(oh, y, (((0,), (0,)), ((), ())),
                         preferred_element_type=jnp.float32)
    s2 = lax.dot_general(oh, y * y, (((0,), (0,)), ((), ())),
                         preferred_element_type=jnp.float32)
    cnt = jnp.sum(oh, axis=0, keepdims=True)

    @pl.when(i == 0)
    def _init():
        s1_ref[...] = jnp.zeros_like(s1_ref)
        s2_ref[...] = jnp.zeros_like(s2_ref)
        cnt_ref[...] = jnp.zeros_like(cnt_ref)

    s1_ref[...] += s1
    s2_ref[...] += s2
    cnt_ref[...] += cnt


def _elu_stats(agg, b, batch2d):
    n, c = agg.shape
    grid = n // _BN
    return pl.pallas_call(
        _stats_body,
        grid=(grid,),
        in_specs=[
            pl.BlockSpec((_BN, c), lambda i: (i, 0)),
            pl.BlockSpec((1, c), lambda i: (0, 0)),
            pl.BlockSpec((_BN, 1), lambda i: (i, 0)),
        ],
        out_specs=[
            pl.BlockSpec((_BN, c), lambda i: (i, 0)),
            pl.BlockSpec((_G, c), lambda i: (0, 0)),
            pl.BlockSpec((_G, c), lambda i: (0, 0)),
            pl.BlockSpec((1, _G), lambda i: (0, 0)),
        ],
        out_shape=[
            jax.ShapeDtypeStruct((n, c), jnp.float32),
            jax.ShapeDtypeStruct((_G, c), jnp.float32),
            jax.ShapeDtypeStruct((_G, c), jnp.float32),
            jax.ShapeDtypeStruct((1, _G), jnp.float32),
        ],
        interpret=_INTERPRET,
    )(agg, b, batch2d)


def _norm_body(y_ref, mean_ref, rstd_ref, w_ref, b_ref, pw_ref, batch_ref,
               xn_ref, sc_ref):
    bb = batch_ref[...]
    oh = (bb == lax.broadcasted_iota(jnp.int32, (1, _G), 1)).astype(jnp.float32)
    mean = lax.dot_general(oh, mean_ref[...], (((1,), (0,)), ((), ())),
                           preferred_element_type=jnp.float32)
    rstd = lax.dot_general(oh, rstd_ref[...], (((1,), (0,)), ((), ())),
                           preferred_element_type=jnp.float32)
    xn = (y_ref[...] - mean) * rstd * w_ref[...] + b_ref[...]
    xn_ref[...] = xn
    pw = pw_ref[...]
    sc_ref[...] = jnp.tanh(jnp.dot(xn, pw, preferred_element_type=jnp.float32))


def _inorm_apply(y, meanG, rstdG, w, b, pw_scaled, batch2d):
    n, c = y.shape
    grid = n // _BN
    return pl.pallas_call(
        _norm_body,
        grid=(grid,),
        in_specs=[
            pl.BlockSpec((_BN, c), lambda i: (i, 0)),
            pl.BlockSpec((_G, c), lambda i: (0, 0)),
            pl.BlockSpec((_G, c), lambda i: (0, 0)),
            pl.BlockSpec((1, c), lambda i: (0, 0)),
            pl.BlockSpec((1, c), lambda i: (0, 0)),
            pl.BlockSpec((c, 1), lambda i: (0, 0)),
            pl.BlockSpec((_BN, 1), lambda i: (i, 0)),
        ],
        out_specs=[
            pl.BlockSpec((_BN, c), lambda i: (i, 0)),
            pl.BlockSpec((_BN, 1), lambda i: (i, 0)),
        ],
        out_shape=[
            jax.ShapeDtypeStruct((n, c), jnp.float32),
            jax.ShapeDtypeStruct((n, 1), jnp.float32),
        ],
        interpret=_INTERPRET,
    )(y, meanG, rstdG, w, b, pw_scaled, batch2d)


def _gate_body(xn_ref, gw1_ref, gb1_ref, gw2_ref, gb2_ref, nw_ref, nb_ref,
               batch_ref, gate_ref, xt_ref, m_ref):
    i = pl.program_id(0)
    xn = xn_ref[...]
    g1 = _elu(jnp.dot(xn, gw1_ref[...], preferred_element_type=jnp.float32)
              + gb1_ref[...])
    gate = jnp.dot(g1, gw2_ref[...], preferred_element_type=jnp.float32) \
        + gb2_ref[...]
    gate_ref[...] = gate
    xt_ref[...] = _elu(jnp.dot(xn, nw_ref[...],
                               preferred_element_type=jnp.float32) + nb_ref[...])
    bb = batch_ref[...]
    oh = bb == lax.broadcasted_iota(jnp.int32, (1, _G), 1)
    gm = jnp.max(jnp.where(oh, gate, -1e30), axis=0, keepdims=True)

    @pl.when(i == 0)
    def _init():
        m_ref[...] = jnp.full_like(m_ref, -1e30)

    m_ref[...] = jnp.maximum(m_ref[...], gm)


def _gate_head(xn, gw1, gb1, gw2, gb2, nw, nb, batch2d):
    n, c = xn.shape
    grid = n // _BN
    return pl.pallas_call(
        _gate_body,
        grid=(grid,),
        in_specs=[
            pl.BlockSpec((_BN, c), lambda i: (i, 0)),
            pl.BlockSpec((c, 128), lambda i: (0, 0)),
            pl.BlockSpec((1, 128), lambda i: (0, 0)),
            pl.BlockSpec((128, 1), lambda i: (0, 0)),
            pl.BlockSpec((1, 1), lambda i: (0, 0)),
            pl.BlockSpec((c, c), lambda i: (0, 0)),
            pl.BlockSpec((1, c), lambda i: (0, 0)),
            pl.BlockSpec((_BN, 1), lambda i: (i, 0)),
        ],
        out_specs=[
            pl.BlockSpec((_BN, 1), lambda i: (i, 0)),
            pl.BlockSpec((_BN, c), lambda i: (i, 0)),
            pl.BlockSpec((1, _G), lambda i: (0, 0)),
        ],
        out_shape=[
            jax.ShapeDtypeStruct((n, 1), jnp.float32),
            jax.ShapeDtypeStruct((n, c), jnp.float32),
            jax.ShapeDtypeStruct((1, _G), jnp.float32),
        ],
        interpret=_INTERPRET,
    )(xn, gw1, gb1, gw2, gb2, nw, nb, batch2d)


def _pool_body(gate_ref, xt_ref, m_ref, batch_ref, s_ref, t_ref):
    i = pl.program_id(0)
    bb = batch_ref[...]
    ohb = bb == lax.broadcasted_iota(jnp.int32, (1, _G), 1)
    oh = ohb.astype(jnp.float32)
    m = jnp.max(jnp.where(ohb, m_ref[...], -jnp.inf), axis=1, keepdims=True)
    m = jnp.where(jnp.isfinite(m), m, 0.0)
    e = jnp.exp(gate_ref[...] - m)
    s = jnp.sum(oh * e, axis=0, keepdims=True)
    t = lax.dot_general(oh * e, xt_ref[...], (((0,), (0,)), ((), ())),
                        preferred_element_type=jnp.float32)

    @pl.when(i == 0)
    def _init():
        s_ref[...] = jnp.zeros_like(s_ref)
        t_ref[...] = jnp.zeros_like(t_ref)

    s_ref[...] += s
    t_ref[...] += t


def _gate_pool(gate, xt, m, batch2d):
    n, c = xt.shape
    grid = n // _BN
    return pl.pallas_call(
        _pool_body,
        grid=(grid,),
        in_specs=[
            pl.BlockSpec((_BN, 1), lambda i: (i, 0)),
            pl.BlockSpec((_BN, c), lambda i: (i, 0)),
            pl.BlockSpec((1, _G), lambda i: (0, 0)),
            pl.BlockSpec((_BN, 1), lambda i: (i, 0)),
        ],
        out_specs=[
            pl.BlockSpec((1, _G), lambda i: (0, 0)),
            pl.BlockSpec((_G, c), lambda i: (0, 0)),
        ],
        out_shape=[
            jax.ShapeDtypeStruct((1, _G), jnp.float32),
            jax.ShapeDtypeStruct((_G, c), jnp.float32),
        ],
        interpret=_INTERPRET,
    )(gate, xt, m, batch2d)


def _head_body(t_ref, s_ref, w1_ref, b1_ref, w2_ref, b2_ref, w3_ref, b3_ref,
               out_ref):
    s = s_ref[...]
    xg = t_ref[...] / (s + 1e-16)
    h = _elu(jnp.dot(xg, w1_ref[...], preferred_element_type=jnp.float32)
             + b1_ref[...])
    h = _elu(jnp.dot(h, w2_ref[...], preferred_element_type=jnp.float32)
             + b2_ref[...])
    o = jnp.dot(h, w3_ref[...], preferred_element_type=jnp.float32) + b3_ref[...]
    mx = jnp.max(o, axis=1, keepdims=True)
    z = o - mx
    lse = jnp.log(jnp.sum(jnp.exp(z), axis=1, keepdims=True))
    out_ref[...] = z - lse


def _classifier(T, sg, l1w, l1b, l2w, l2b, l3w, l3b):
    return pl.pallas_call(
        _head_body,
        out_shape=jax.ShapeDtypeStruct((_G, _NC), jnp.float32),
        interpret=_INTERPRET,
    )(T, sg, l1w, l1b, l2w, l2b, l3w, l3b)


_BE = 1000


def _scan_body(hs_ref, al_ref, m_ref, tri_ref, p_ref, pe_ref, th_ref, te_ref):
    e = jnp.exp(al_ref[...] - m_ref[...])
    oc = hs_ref.shape[1] // 2
    e_exp = jnp.concatenate(
        [jnp.broadcast_to(e[:, :1], (_BE, oc)),
         jnp.broadcast_to(e[:, 1:2], (_BE, oc))], axis=1)
    hv = hs_ref[...] * e_exp
    tri = tri_ref[...]
    p_ref[...] = jnp.dot(tri, hv, preferred_element_type=jnp.float32)
    pe_ref[...] = jnp.dot(tri, e, preferred_element_type=jnp.float32)
    th_ref[...] = jnp.sum(hv, axis=0, keepdims=True).reshape(1, 1, 2 * oc)
    te_ref[...] = jnp.sum(e, axis=0, keepdims=True).reshape(1, 1, 2)


def _edge_scan(hs, al_e, m_e, tri):
    e2, c = hs.shape
    nb = e2 // _BE
    return pl.pallas_call(
        _scan_body,
        grid=(nb,),
        in_specs=[
            pl.BlockSpec((_BE, c), lambda i: (i, 0)),
            pl.BlockSpec((_BE, 2), lambda i: (i, 0)),
            pl.BlockSpec((_BE, 2), lambda i: (i, 0)),
            pl.BlockSpec((_BE, _BE), lambda i: (0, 0)),
        ],
        out_specs=[
            pl.BlockSpec((_BE, c), lambda i: (i, 0)),
            pl.BlockSpec((_BE, 2), lambda i: (i, 0)),
            pl.BlockSpec((1, 1, c), lambda i: (i, 0, 0)),
            pl.BlockSpec((1, 1, 2), lambda i: (i, 0, 0)),
        ],
        out_shape=[
            jax.ShapeDtypeStruct((e2, c), jnp.float32),
            jax.ShapeDtypeStruct((e2, 2), jnp.float32),
            jax.ShapeDtypeStruct((nb, 1, c), jnp.float32),
            jax.ShapeDtypeStruct((nb, 1, 2), jnp.float32),
        ],
        interpret=_INTERPRET,
    )(hs, al_e, m_e, tri)


def _edge_aggregate_v2(h, als, ald, s, d, n, heads, oc, tri):
    order = jnp.argsort(d)
    ds = d[order]
    ss = s[order]
    al = als[ss] + ald[ds]
    al = jnp.where(al > 0, al, 0.2 * al)
    m = jax.ops.segment_max(al, ds, num_segments=n)
    m = jnp.where(jnp.isfinite(m), m, 0.0)
    hs = h[ss]
    p, pe, th, te = _edge_scan(hs, al, m[ds], tri)

    nb = th.shape[0]
    ch = jnp.concatenate([jnp.zeros((1, h.shape[1]), jnp.float32),
                          jnp.cumsum(th.reshape(nb, -1), axis=0)[:-1]])
    ce = jnp.concatenate([jnp.zeros((1, 2), jnp.float32),
                          jnp.cumsum(te.reshape(nb, -1), axis=0)[:-1]])
    ar = jnp.arange(n, dtype=ds.dtype)
    lo = jnp.searchsorted(ds, ar)
    hi = jnp.searchsorted(ds, ar, side="right")

    def _pref(pos, ploc, cblk):
        idx = jnp.maximum(pos - 1, 0)
        val = ploc[idx] + cblk[idx // _BE]
        return jnp.where((pos > 0)[:, None], val, 0.0)

    seg_h = _pref(hi, p, ch) - _pref(lo, p, ch)
    seg_e = _pref(hi, pe, ce) - _pref(lo, pe, ce)
    seg_e = jnp.concatenate(
        [jnp.broadcast_to(seg_e[:, :1], (n, oc)),
         jnp.broadcast_to(seg_e[:, 1:2], (n, oc))], axis=1)
    return seg_h / (seg_e + 1e-16)


def _head_mat(a):
    heads, oc = a.shape
    cols = []
    for hh in range(heads):
        col = jnp.zeros((heads * oc,), jnp.float32).at[hh * oc:(hh + 1) * oc].set(a[hh])
        cols.append(col)
    return jnp.stack(cols, axis=1)


def _edge_aggregate(h, als, ald, s, d, n, heads, oc):
    al = als[s] + ald[d]
    al = jnp.where(al > 0, al, 0.2 * al)
    m = jax.ops.segment_max(al, d, num_segments=n)
    m = jnp.where(jnp.isfinite(m), m, 0.0)
    e = jnp.exp(al - m[d])
    ssum = jax.ops.segment_sum(e, d, num_segments=n)
    he = h[s].reshape(-1, heads, oc) * e[..., None]
    agg = jax.ops.segment_sum(he, d, num_segments=n)
    return (agg / (ssum + 1e-16)[..., None]).reshape(n, heads * oc)


def kernel(pos, edge_index, batch, W1, a_src1, a_dst1, b1, in1_w, in1_b,
           pool_w, W2, a_src2, a_dst2, b2, in2_w, in2_b, gw1, gb1, gw2, gb2,
           nw, nb, l1w, l1b, l2w, l2b, l3w, l3b):
    n = _N
    batch2d = batch.reshape(n, 1)
    tri = jnp.tril(jnp.ones((_BE, _BE), jnp.float32))

    keep = edge_index[0] != edge_index[1]
    s1 = jnp.where(keep, edge_index[0], n)
    d1 = jnp.where(keep, edge_index[1], n)
    loops = jnp.arange(n, dtype=jnp.int32)
    es = jnp.concatenate([s1, loops])
    ed = jnp.concatenate([d1, loops])

    h1, als1, ald1 = _node_proj(pos, W1, _head_mat(a_src1), _head_mat(a_dst1))
    agg1 = _edge_aggregate_v2(h1, als1, ald1, es, ed, n, 2, 32, tri)

    y1, S1, S2, cnt = _elu_stats(agg1, b1.reshape(1, -1), batch2d)
    cnt_g = jnp.maximum(cnt.reshape(_G, 1), 1.0)
    mean1 = S1 / cnt_g
    var1 = jnp.maximum(S2 / cnt_g - mean1 * mean1, 0.0)
    rstd1 = 1.0 / jnp.sqrt(var1 + 1e-5)
    pw = (pool_w / (jnp.linalg.norm(pool_w) + 1e-16)).reshape(-1, 1)
    x1, score = _inorm_apply(y1, mean1, rstd1, in1_w.reshape(1, -1),
                             in1_b.reshape(1, -1), pw, batch2d)
    score = score[:, 0]

    counts = cnt.reshape(_G)
    k = jnp.ceil(_RATIO * counts).astype(jnp.int32)
    order = jnp.lexsort((-score, batch))
    starts = jnp.concatenate([jnp.zeros((1,), jnp.int32),
                              jnp.cumsum(counts.astype(jnp.int32))[:-1]])
    pr = jnp.arange(n, dtype=jnp.int32) - starts[batch[order]]
    rank = jnp.zeros_like(pr).at[order].set(pr)
    mask = rank < k[batch]
    perm = jnp.nonzero(mask, size=n, fill_value=n)[0]
    m_nodes = jnp.sum(mask.astype(jnp.int32))
    valid = jnp.arange(n, dtype=jnp.int32) < m_nodes
    batch2 = jnp.where(valid, batch[perm], _G).astype(jnp.int32)
    emask = mask[edge_index[0]] & mask[edge_index[1]] & keep
    newid = jnp.cumsum(mask.astype(jnp.int32)) - 1
    src2 = jnp.where(emask, newid[edge_index[0]], n)
    dst2 = jnp.where(emask, newid[edge_index[1]], n)
    lid = jnp.where(valid, loops, n)
    es2 = jnp.concatenate([src2, lid])
    ed2 = jnp.concatenate([dst2, lid])

    xin2 = jnp.where(valid[:, None], x1[perm] * score[perm][:, None], 0.0)

    h2, als2, ald2 = _node_proj(xin2, W2, _head_mat(a_src2), _head_mat(a_dst2))
    agg2 = _edge_aggregate_v2(h2, als2, ald2, es2, ed2, n, 2, 128, tri)

    batch2d2 = batch2.reshape(n, 1)
    y2, S1b, S2b, cntb = _elu_stats(agg2, b2.reshape(1, -1), batch2d2)
    cnt_g2 = jnp.maximum(cntb.reshape(_G, 1), 1.0)
    mean2 = S1b / cnt_g2
    var2 = jnp.maximum(S2b / cnt_g2 - mean2 * mean2, 0.0)
    rstd2 = 1.0 / jnp.sqrt(var2 + 1e-5)
    zero_pw = jnp.zeros((256, 1), jnp.float32)
    x2, _ = _inorm_apply(y2, mean2, rstd2, in2_w.reshape(1, -1),
                         in2_b.reshape(1, -1), zero_pw, batch2d2)

    gate, xt, mg = _gate_head(x2, gw1, gb1.reshape(1, -1), gw2,
                              gb2.reshape(1, 1), nw, nb.reshape(1, -1),
                              batch2d2)
    sG, T = _gate_pool(gate, xt, mg, batch2d2)
    return _classifier(T, sG.reshape(_G, 1), l1w, l1b.reshape(1, -1),
                       l2w, l2b.reshape(1, -1), l3w, l3b.reshape(1, -1))
